# Initial kernel scaffold; baseline (speedup 1.0000x reference)
import jax
import jax.numpy as jnp
from jax.experimental import pallas as pl


def kernel(batch, landmarks, patch_size):
    raise NotImplementedError("write your pallas kernel here")


# speedup vs baseline: 23.6388x; 23.6388x over previous
import functools

import jax
import jax.numpy as jnp
from jax import lax
from jax.experimental import pallas as pl
from jax.experimental.pallas import tpu as pltpu
from jax.experimental.pallas import tpu_sc as plsc

_NC, _NS, _L = 2, 16, 16

_DNUMS = lax.GatherDimensionNumbers(
    offset_dims=(), collapsed_slice_dims=(0,), start_index_map=(0,))


def _dg(v, idx):
    return lax.gather(v, idx[:, None], _DNUMS, (1,),
                      mode=lax.GatherScatterMode.PROMISE_IN_BOUNDS)


def _ifloor(v):
    t = v.astype(jnp.int32)
    return jnp.where(t.astype(jnp.float32) > v, t - 1, t)


@functools.lru_cache(maxsize=None)
def _build(B, C, H, W, n_lm):
    nch = W // _L
    rows_used = n_lm * C * 2
    rows_per_b = (rows_used + 7) // 8 * 8
    mesh = plsc.VectorSubcoreMesh(
        core_axis_name="c", subcore_axis_name="s",
        num_cores=_NC, num_subcores=_NS)

    @functools.partial(
        pl.kernel,
        out_type=jax.ShapeDtypeStruct((B * rows_per_b, 128), jnp.float32),
        mesh=mesh,
        scratch_types=[
            pltpu.VMEM((H, W), jnp.float32),
            pltpu.VMEM((n_lm, _L), jnp.float32),
            pltpu.VMEM((n_lm, _L), jnp.float32),
            pltpu.VMEM((rows_per_b, 128), jnp.float32),
        ],
    )
    def kern(img_hbm, xbc_hbm, ybc_hbm, out_hbm, plane, xv, yv, slab):
        b = lax.axis_index("s") * _NC + lax.axis_index("c")
        pltpu.sync_copy(xbc_hbm.at[b], xv)
        pltpu.sync_copy(ybc_hbm.at[b], yv)
        iota = lax.iota(jnp.int32, _L)

        for ch in range(C):
            pltpu.sync_copy(img_hbm.at[b, ch], plane)

            def lm_body(i, carry):
                axv = xv[i]
                ayv = yv[i]
                bxv = _ifloor(axv)
                byv = _ifloor(ayv)
                fxv = axv - bxv.astype(jnp.float32)
                fyv = ayv - byv.astype(jnp.float32)
                bx_s = bxv[0]
                by_s = byv[0]

                ca = jnp.clip(lax.div(bx_s - 15, 16), 0, nch - 1)
                s0 = pl.multiple_of(ca * 16, 16)
                s1 = pl.multiple_of(jnp.minimum(ca + 1, nch - 1) * 16, 16)
                cav = jnp.clip(lax.div(bxv - 15, 16), 0, nch - 1) * 16

                col0 = bxv - iota
                col1 = col0 + 1
                lc0 = jnp.clip(col0 - cav, 0, 2 * _L - 1)
                lc1 = jnp.clip(col1 - cav, 0, 2 * _L - 1)
                a0 = jnp.where((col0 >= 0) & (col0 <= W - 1), 1.0 - fxv, 0.0)
                a1 = jnp.where((col1 >= 0) & (col1 <= W - 1), fxv, 0.0)
                sel0 = lc0 < _L
                sel1 = lc1 < _L
                i0 = lc0 & (_L - 1)
                i1 = lc1 & (_L - 1)

                def rowcomb(r):
                    rg = jnp.clip(by_s + r, 0, H - 1)
                    v0 = plane[rg, pl.ds(s0, _L)]
                    v1 = plane[rg, pl.ds(s1, _L)]
                    g0 = jnp.where(sel0, _dg(v0, i0), _dg(v1, i0))
                    g1 = jnp.where(sel1, _dg(v0, i1), _dg(v1, i1))
                    rw = jnp.where((byv + r >= 0) & (byv + r <= H - 1), 1.0, 0.0)
                    return (a0 * g0 + a1 * g1) * rw

                base = (C * 2) * i + 2 * ch
                rc_prev = rowcomb(0)
                for pi in range(16):
                    rc_cur = rowcomb(pi + 1)
                    slab[base + pi // 8, pl.ds((pi % 8) * 16, _L)] = (
                        (1.0 - fyv) * rc_prev + fyv * rc_cur)
                    rc_prev = rc_cur
                return carry

            lax.fori_loop(0, n_lm, lm_body, jnp.int32(0))

        pltpu.sync_copy(slab, out_hbm.at[pl.ds(b * rows_per_b, rows_per_b)])

    return kern


def kernel(batch, landmarks, patch_size):
    B, C, H, W = batch.shape
    n_lm = landmarks.shape[1] // 2
    half = patch_size / 2.0
    lm = landmarks.reshape(B, n_lm, 2)
    ax = lm[..., 0].astype(jnp.float32) + (half - 0.5)
    ay = lm[..., 1].astype(jnp.float32) - (half + 0.5)
    xbc = jnp.broadcast_to(ax[..., None], (B, n_lm, _L))
    ybc = jnp.broadcast_to(ay[..., None], (B, n_lm, _L))
    out = _build(B, C, H, W, n_lm)(batch, xbc, ybc)
    rows_used = n_lm * C * 2
    rows_per_b = (rows_used + 7) // 8 * 8
    out = out.reshape(B, rows_per_b, 128)[:, :rows_used]
    return out.reshape(B, n_lm, C, 16, 16)
